# baseline (speedup 1.0000x reference)
import functools

import jax
import jax.numpy as jnp
from jax.experimental import pallas as pl
from jax.experimental.pallas import tpu as pltpu

TOPK = 64


def _enc_body(x_ref, w_ref, b_ref, out_ref):
    acc = jnp.dot(x_ref[...], w_ref[...], preferred_element_type=jnp.float32)
    out_ref[...] = acc + b_ref[...]


def _encode(x, W_enc, b_enc):
    B, K = x.shape
    H = W_enc.shape[1]
    bm, bh = 256, 2048
    grid = (B // bm, H // bh)
    return pl.pallas_call(
        _enc_body,
        grid=grid,
        in_specs=[
            pl.BlockSpec((bm, K), lambda i, j: (i, 0)),
            pl.BlockSpec((K, bh), lambda i, j: (0, j)),
            pl.BlockSpec((bh,), lambda i, j: (j,)),
        ],
        out_specs=pl.BlockSpec((bm, bh), lambda i, j: (i, j)),
        out_shape=jax.ShapeDtypeStruct((B, H), jnp.float32),
    )(x, W_enc, b_enc)


def kernel(x, W_enc, W_dec, b_enc, b_dec):
    preact = _encode(x, W_enc, b_enc)
    hidden = jax.nn.relu(preact)
    values, indices = jax.lax.top_k(hidden, TOPK)
    rows = jnp.arange(hidden.shape[0])[:, None]
    hidden_sparse = jnp.zeros_like(hidden).at[rows, indices].set(values)
    out = jax.nn.relu(hidden_sparse @ W_dec + b_dec)
    return (out, hidden_sparse, preact)


# speedup vs baseline: 11.3437x; 11.3437x over previous
import jax
import jax.numpy as jnp
from jax.experimental import pallas as pl

TOPK = 64


def _enc_body(x_ref, w_ref, b_ref, out_ref):
    acc = jnp.dot(x_ref[...], w_ref[...], preferred_element_type=jnp.float32)
    out_ref[...] = acc + b_ref[...]


def _encode(x, W_enc, b_enc):
    B, K = x.shape
    H = W_enc.shape[1]
    bh = 1024
    return pl.pallas_call(
        _enc_body,
        grid=(H // bh,),
        in_specs=[
            pl.BlockSpec((B, K), lambda j: (0, 0)),
            pl.BlockSpec((K, bh), lambda j: (0, j)),
            pl.BlockSpec((bh,), lambda j: (j,)),
        ],
        out_specs=pl.BlockSpec((B, bh), lambda j: (0, j)),
        out_shape=jax.ShapeDtypeStruct((B, H), jnp.float32),
    )(x, W_enc, b_enc)


def _thresh_body(pre_ref, hs_ref, t_ref):
    h = jnp.maximum(pre_ref[...], 0.0)
    bits = jax.lax.bitcast_convert_type(h, jnp.int32)
    rows = h.shape[0]
    lo0 = jnp.zeros((rows, 1), jnp.int32)
    hi0 = jnp.full((rows, 1), jnp.int32(0x7FFFFFFF))

    def it(_, carry):
        lo, hi = carry
        mid = lo + (hi - lo) // 2
        cnt = jnp.sum((bits >= mid).astype(jnp.int32), axis=1, keepdims=True)
        ge = cnt >= TOPK
        return jnp.where(ge, mid, lo), jnp.where(ge, hi, mid)

    lo, _ = jax.lax.fori_loop(0, 31, it, (lo0, hi0))
    t = jax.lax.bitcast_convert_type(lo, jnp.float32)
    hs_ref[...] = jnp.where(h >= t, h, 0.0)
    t_ref[...] = t


def _threshold_mask(preact):
    B, H = preact.shape
    bm = 128
    return pl.pallas_call(
        _thresh_body,
        grid=(B // bm,),
        in_specs=[pl.BlockSpec((bm, H), lambda i: (i, 0))],
        out_specs=[
            pl.BlockSpec((bm, H), lambda i: (i, 0)),
            pl.BlockSpec((bm, 1), lambda i: (i, 0)),
        ],
        out_shape=[
            jax.ShapeDtypeStruct((B, H), jnp.float32),
            jax.ShapeDtypeStruct((B, 1), jnp.float32),
        ],
    )(preact)


def _dec_body(hs_ref, w_ref, b_ref, out_ref):
    k = pl.program_id(1)
    nk = pl.num_programs(1)
    acc = jnp.dot(hs_ref[...], w_ref[...], preferred_element_type=jnp.float32)

    @pl.when(k == 0)
    def _():
        out_ref[...] = acc + b_ref[...]

    @pl.when(k != 0)
    def _():
        out_ref[...] = out_ref[...] + acc

    @pl.when(k == nk - 1)
    def _():
        out_ref[...] = jnp.maximum(out_ref[...], 0.0)


def _decode(hs, W_dec, b_dec):
    B, H = hs.shape
    D = W_dec.shape[1]
    bj, bk = 1024, 2048
    return pl.pallas_call(
        _dec_body,
        grid=(D // bj, H // bk),
        in_specs=[
            pl.BlockSpec((B, bk), lambda j, k: (0, k)),
            pl.BlockSpec((bk, bj), lambda j, k: (k, j)),
            pl.BlockSpec((bj,), lambda j, k: (j,)),
        ],
        out_specs=pl.BlockSpec((B, bj), lambda j, k: (0, j)),
        out_shape=jax.ShapeDtypeStruct((B, D), jnp.float32),
    )(hs, W_dec, b_dec)


def kernel(x, W_enc, W_dec, b_enc, b_dec):
    preact = _encode(x, W_enc, b_enc)
    hidden_sparse, _t = _threshold_mask(preact)
    out = _decode(hidden_sparse, W_dec, b_dec)
    return (out, hidden_sparse, preact)
